# baseline (speedup 1.0000x reference)
import functools

import jax
import jax.numpy as jnp
from jax import lax
from jax.experimental import pallas as pl
from jax.experimental.pallas import tpu as pltpu
from jax.experimental.pallas import tpu_sc as plsc

_N = 10000
_D = 128
_E = 320000

_NC = 2
_NS = 16
_NW = _NC * _NS
_CHUNK = 128
_G = 40
_NSUPER = 2
_NCHUNK = _G * _NSUPER
_EPW = _CHUNK * _NCHUNK
_EPAD = _NW * _EPW
_NBUF = 2
_NPAD = 10240
_RPS = _NPAD // _NS


def _sc_aggregate(x, src, dst, zblk):
    mesh = plsc.VectorSubcoreMesh(core_axis_name="c", subcore_axis_name="s")

    @functools.partial(
        pl.kernel,
        out_type=jax.ShapeDtypeStruct((_NC, _NPAD, _D), jnp.float32),
        mesh=mesh,
        scratch_types=[
            pltpu.VMEM((_G, _CHUNK), jnp.int32),
            pltpu.VMEM((_G, _CHUNK), jnp.int32),
        ] + [pltpu.VMEM((_CHUNK, _D), jnp.float32) for _ in range(_NBUF)]
          + [pltpu.VMEM_SHARED((_NPAD, _D), jnp.float32)]
          + [pltpu.SemaphoreType.DMA for _ in range(2 * _NBUF)],
    )
    def agg_kernel(x_hbm, src_hbm, dst_hbm, z_hbm, out_hbm,
                   sidx, didx, *rest):
        bufs = rest[:_NBUF]
        accsh = rest[_NBUF]
        gsems = rest[_NBUF + 1:2 * _NBUF + 1]
        ssems = rest[2 * _NBUF + 1:]
        cid = lax.axis_index("c")
        sid = lax.axis_index("s")
        wid = sid * _NC + cid

        pltpu.sync_copy(z_hbm, accsh.at[pl.ds(sid * _RPS, _RPS)])
        plsc.subcore_barrier()

        def outer(g, _):
            pltpu.sync_copy(src_hbm.at[wid, g], sidx)
            pltpu.sync_copy(dst_hbm.at[wid, g], didx)

            def body(i, _):
                j = i * _NBUF
                gd, sd = [], []
                for b in range(_NBUF):
                    gd.append(pltpu.async_copy(x_hbm.at[sidx.at[j + b]],
                                               bufs[b], gsems[b]))
                for b in range(_NBUF):
                    gd[b].wait()
                    sd.append(pltpu.async_copy(
                        bufs[b], accsh.at[didx.at[j + b]], ssems[b], add=True))
                for b in range(_NBUF):
                    sd[b].wait()
                return ()

            lax.fori_loop(0, _G // _NBUF, body, (), unroll=False)
            return ()

        lax.fori_loop(0, _NSUPER, outer, (), unroll=False)
        plsc.subcore_barrier()

        pltpu.sync_copy(accsh.at[pl.ds(sid * _RPS, _RPS)],
                        out_hbm.at[cid, pl.ds(sid * _RPS, _RPS)])

    return agg_kernel(x, src, dst, zblk)


def _tc_mlp(x, aggp, W1, b1, g1, bb1, W2, b2, g2, bb2, lng, lnb):
    def body(x_ref, ap_ref, w1_ref, b1_ref, g1_ref, bb1_ref,
             w2_ref, b2_ref, g2_ref, bb2_ref, lng_ref, lnb_ref, o_ref):
        xx = x_ref[...]
        h = xx + ap_ref[0, :_N] + ap_ref[1, :_N]
        h = lax.dot_general(h, w1_ref[...], (((1,), (1,)), ((), ())),
                            preferred_element_type=jnp.float32) + b1_ref[...]
        mu = jnp.mean(h, axis=0, keepdims=True)
        var = jnp.mean(jnp.square(h - mu), axis=0, keepdims=True)
        h = (h - mu) * lax.rsqrt(var + 1e-5) * g1_ref[...] + bb1_ref[...]
        h = jnp.maximum(h, 0.0)
        h = lax.dot_general(h, w2_ref[...], (((1,), (1,)), ((), ())),
                            preferred_element_type=jnp.float32) + b2_ref[...]
        mu = jnp.mean(h, axis=0, keepdims=True)
        var = jnp.mean(jnp.square(h - mu), axis=0, keepdims=True)
        h = (h - mu) * lax.rsqrt(var + 1e-5) * g2_ref[...] + bb2_ref[...]
        h = jnp.maximum(h, 0.0)
        o = h + xx
        mu = jnp.mean(o, axis=1, keepdims=True)
        var = jnp.mean(jnp.square(o - mu), axis=1, keepdims=True)
        o_ref[...] = (o - mu) * lax.rsqrt(var + 1e-5) * lng_ref[...] + lnb_ref[...]

    return pl.pallas_call(
        body,
        out_shape=jax.ShapeDtypeStruct((_N, _D), jnp.float32),
    )(x, aggp, W1, b1, g1, bb1, W2, b2, g2, bb2, lng, lnb)


def kernel(x, edge_index, W1, b1, bn1_g, bn1_b, W2, b2, bn2_g, bn2_b, ln_g, ln_b):
    npad = _EPAD - _E
    x2 = jnp.zeros((_NC * _NPAD, _D), jnp.float32)
    x2 = x2.at[:_N].set(x).at[_NPAD:_NPAD + _N].set(x)
    src = jnp.concatenate([edge_index[0], jnp.zeros((npad,), jnp.int32)])
    trash = _N + (jnp.arange(npad, dtype=jnp.int32) % (_NPAD - _N))
    dst = jnp.concatenate([edge_index[1], trash])
    src = src.reshape(_NW, _NSUPER, _G, _CHUNK)
    src = src + ((jnp.arange(_NW, dtype=jnp.int32) % _NC)
                 * _NPAD).reshape(_NW, 1, 1, 1)
    dst = dst.reshape(_NW, _NSUPER, _G, _CHUNK)
    zblk = jnp.zeros((_RPS, _D), jnp.float32)
    aggp = _sc_aggregate(x2, src, dst, zblk)
    r = lambda v: v.reshape(1, _D)
    return _tc_mlp(x, aggp, W1, r(b1), r(bn1_g), r(bn1_b),
                   W2, r(b2), r(bn2_g), r(bn2_b), r(ln_g), r(ln_b))


# speedup vs baseline: 3.3410x; 3.3410x over previous
import functools

import jax
import jax.numpy as jnp
from jax import lax
from jax.experimental import pallas as pl
from jax.experimental.pallas import tpu as pltpu
from jax.experimental.pallas import tpu_sc as plsc

_N = 10000
_D = 128
_E = 320000

_NC = 2
_NS = 16
_NW = _NC * _NS
_CHUNK = 125
_G = 40
_NSUPER = 2
_NCHUNK = _G * _NSUPER
_EPW = _CHUNK * _NCHUNK
_NBUF = 2
_NPAD = 10240
_RPS = _NPAD // _NS


def _sc_aggregate(x, src, dst, zblk):
    mesh = plsc.VectorSubcoreMesh(core_axis_name="c", subcore_axis_name="s")

    @functools.partial(
        pl.kernel,
        out_type=jax.ShapeDtypeStruct((_NC, _NPAD, _D), jnp.float32),
        mesh=mesh,
        scratch_types=[
            pltpu.VMEM((_G, _CHUNK), jnp.int32),
            pltpu.VMEM((_G, _CHUNK), jnp.int32),
        ] + [pltpu.VMEM((_CHUNK, _D), jnp.float32) for _ in range(_NBUF)]
          + [pltpu.VMEM_SHARED((_NPAD, _D), jnp.float32)]
          + [pltpu.SemaphoreType.DMA for _ in range(2 * _NBUF)],
    )
    def agg_kernel(x_hbm, src_hbm, dst_hbm, z_hbm, out_hbm,
                   sidx, didx, *rest):
        bufs = rest[:_NBUF]
        accsh = rest[_NBUF]
        gsems = rest[_NBUF + 1:2 * _NBUF + 1]
        ssems = rest[2 * _NBUF + 1:]
        cid = lax.axis_index("c")
        sid = lax.axis_index("s")
        wid = sid * _NC + cid

        pltpu.sync_copy(z_hbm, accsh.at[pl.ds(sid * _RPS, _RPS)])
        plsc.subcore_barrier()

        def outer(g, _):
            pltpu.sync_copy(src_hbm.at[wid, g], sidx)
            pltpu.sync_copy(dst_hbm.at[wid, g], didx)

            def body(i, _):
                j = i * _NBUF
                gd, sd = [], []
                for b in range(_NBUF):
                    gd.append(pltpu.async_copy(x_hbm.at[sidx.at[j + b]],
                                               bufs[b], gsems[b]))
                for b in range(_NBUF):
                    gd[b].wait()
                    sd.append(pltpu.async_copy(
                        bufs[b], accsh.at[didx.at[j + b]], ssems[b], add=True))
                for b in range(_NBUF):
                    sd[b].wait()
                return ()

            lax.fori_loop(0, _G // _NBUF, body, (), unroll=False)
            return ()

        lax.fori_loop(0, _NSUPER, outer, (), unroll=False)
        plsc.subcore_barrier()

        pltpu.sync_copy(accsh.at[pl.ds(sid * _RPS, _RPS)],
                        out_hbm.at[cid, pl.ds(sid * _RPS, _RPS)])

    return agg_kernel(x, src, dst, zblk)


def _tc_mlp(x, aggp, W1, b1, g1, bb1, W2, b2, g2, bb2, lng, lnb):
    def body(x_ref, ap_ref, w1_ref, b1_ref, g1_ref, bb1_ref,
             w2_ref, b2_ref, g2_ref, bb2_ref, lng_ref, lnb_ref, o_ref):
        xx = x_ref[...]
        h = xx + ap_ref[0, :_N] + ap_ref[1, :_N]
        h = lax.dot_general(h, w1_ref[...], (((1,), (1,)), ((), ())),
                            preferred_element_type=jnp.float32) + b1_ref[...]
        mu = jnp.mean(h, axis=0, keepdims=True)
        var = jnp.mean(jnp.square(h - mu), axis=0, keepdims=True)
        h = (h - mu) * lax.rsqrt(var + 1e-5) * g1_ref[...] + bb1_ref[...]
        h = jnp.maximum(h, 0.0)
        h = lax.dot_general(h, w2_ref[...], (((1,), (1,)), ((), ())),
                            preferred_element_type=jnp.float32) + b2_ref[...]
        mu = jnp.mean(h, axis=0, keepdims=True)
        var = jnp.mean(jnp.square(h - mu), axis=0, keepdims=True)
        h = (h - mu) * lax.rsqrt(var + 1e-5) * g2_ref[...] + bb2_ref[...]
        h = jnp.maximum(h, 0.0)
        o = h + xx
        mu = jnp.mean(o, axis=1, keepdims=True)
        var = jnp.mean(jnp.square(o - mu), axis=1, keepdims=True)
        o_ref[...] = (o - mu) * lax.rsqrt(var + 1e-5) * lng_ref[...] + lnb_ref[...]

    return pl.pallas_call(
        body,
        out_shape=jax.ShapeDtypeStruct((_N, _D), jnp.float32),
    )(x, aggp, W1, b1, g1, bb1, W2, b2, g2, bb2, lng, lnb)


def kernel(x, edge_index, W1, b1, bn1_g, bn1_b, W2, b2, bn2_g, bn2_b, ln_g, ln_b):
    src = edge_index[0].reshape(_NW, _NSUPER, _G, _CHUNK)
    dst = edge_index[1].reshape(_NW, _NSUPER, _G, _CHUNK)
    zblk = jnp.zeros((_RPS, _D), jnp.float32)
    aggp = _sc_aggregate(x, src, dst, zblk)
    r = lambda v: v.reshape(1, _D)
    return _tc_mlp(x, aggp, W1, r(b1), r(bn1_g), r(bn1_b),
                   W2, r(b2), r(bn2_g), r(bn2_b), r(ln_g), r(ln_b))
